# baseline (speedup 1.0000x reference)
import functools

import jax
import jax.numpy as jnp
from jax import lax
from jax.experimental import pallas as pl
from jax.experimental.pallas import tpu as pltpu
from jax.experimental.pallas import tpu_sc as plsc

N = 10000
NP = 10112
H = 128
TASKS = 128
G = 512
E = 320000
CHUNK = 128
NCHUNK = 80
PCH = 40
NW = 32
EP = NW * NCHUNK * CHUNK
VOCAB = 173
VOCAB_PAD = 176
_OFFSETS = (0, 119, 123, 135, 147, 157, 163, 169, 171)
BN = 400
NB = N // BN
RPT = NP // 16


def _mesh():
    return plsc.VectorSubcoreMesh(core_axis_name="c", subcore_axis_name="s",
                                  num_cores=2, num_subcores=16)


def _sc_degree_body(col_hbm, z_hbm, ones_hbm, out_hbm, col_v, ones_v, acc_sh):
    c = lax.axis_index("c")
    s = lax.axis_index("s")
    wid = c * 16 + s
    r0 = s * RPT
    pltpu.sync_copy(z_hbm.at[pl.ds(r0, RPT)], acc_sh.at[pl.ds(r0, RPT)])
    pltpu.sync_copy(col_hbm.at[wid], col_v)
    pltpu.sync_copy(ones_hbm, ones_v)
    plsc.subcore_barrier()

    def body(j, carry):
        pltpu.sync_copy(ones_v, acc_sh.at[col_v.at[j]], add=True)
        return carry

    lax.fori_loop(0, NCHUNK, body, 0)
    plsc.subcore_barrier()
    pltpu.sync_copy(acc_sh.at[pl.ds(r0, RPT)], out_hbm.at[c, pl.ds(r0, RPT)])


@functools.cache
def _sc_degree():
    return pl.kernel(
        _sc_degree_body,
        out_type=jax.ShapeDtypeStruct((2, NP, H), jnp.float32),
        mesh=_mesh(),
        scratch_types=[
            pltpu.VMEM((NCHUNK, CHUNK), jnp.int32),
            pltpu.VMEM((CHUNK, H), jnp.float32),
            pltpu.VMEM_SHARED((NP, H), jnp.float32),
        ],
    )


def _sc_scatter_body(g_hbm, row_hbm, col_hbm, z_hbm, out_hbm,
                     row_v, col_v, msg_a, msg_b, acc_sh, sem_a, sem_b):
    c = lax.axis_index("c")
    s = lax.axis_index("s")
    wid = c * 16 + s
    r0 = s * RPT
    pltpu.sync_copy(z_hbm.at[pl.ds(r0, RPT)], acc_sh.at[pl.ds(r0, RPT)])
    plsc.subcore_barrier()

    def start(j, buf, sem):
        pltpu.async_copy(g_hbm.at[row_v.at[j]], buf, sem)

    def wait(buf, sem):
        pltpu.make_async_copy(g_hbm.at[row_v.at[0]], buf, sem).wait()

    for p in range(NCHUNK // PCH):
        pltpu.sync_copy(row_hbm.at[wid, pl.ds(p * PCH, PCH)], row_v)
        pltpu.sync_copy(col_hbm.at[wid, pl.ds(p * PCH, PCH)], col_v)
        start(0, msg_a, sem_a)

        def body(i, carry):
            wait(msg_a, sem_a)
            start(2 * i + 1, msg_b, sem_b)
            pltpu.sync_copy(msg_a, acc_sh.at[col_v.at[2 * i]], add=True)
            wait(msg_b, sem_b)

            @pl.when(2 * i + 2 < PCH)
            def _():
                start(2 * i + 2, msg_a, sem_a)

            pltpu.sync_copy(msg_b, acc_sh.at[col_v.at[2 * i + 1]], add=True)
            return carry

        lax.fori_loop(0, PCH // 2, body, 0)
    plsc.subcore_barrier()
    pltpu.sync_copy(acc_sh.at[pl.ds(r0, RPT)], out_hbm.at[c, pl.ds(r0, RPT)])


@functools.cache
def _sc_scatter():
    return pl.kernel(
        _sc_scatter_body,
        out_type=jax.ShapeDtypeStruct((2, NP, H), jnp.float32),
        mesh=_mesh(),
        scratch_types=[
            pltpu.VMEM((PCH, CHUNK), jnp.int32),
            pltpu.VMEM((PCH, CHUNK), jnp.int32),
            pltpu.VMEM((CHUNK, H), jnp.float32),
            pltpu.VMEM((CHUNK, H), jnp.float32),
            pltpu.VMEM_SHARED((NP, H), jnp.float32),
            pltpu.SemaphoreType.DMA,
            pltpu.SemaphoreType.DMA,
        ],
    )


def _tc_embed_body(x_ref, tab_ref, dega_ref, degb_ref, w_ref, g_ref, d_ref):
    idx = x_ref[:]
    iota = lax.broadcasted_iota(jnp.int32, (BN, VOCAB_PAD), 1)
    p = jnp.zeros((BN, VOCAB_PAD), jnp.float32)
    for j in range(9):
        p = p + (iota == idx[:, j:j + 1]).astype(jnp.float32)
    h = jnp.dot(p, tab_ref[:], preferred_element_type=jnp.float32)
    deg = dega_ref[:, 0:1] + degb_ref[:, 0:1] + 1.0
    d = lax.rsqrt(deg)
    g_ref[:] = d * jnp.dot(h, w_ref[:], preferred_element_type=jnp.float32)
    d_ref[:] = d


def _tc_embed(x, tab, dega, degb, w1):
    return pl.pallas_call(
        _tc_embed_body,
        grid=(NB,),
        in_specs=[
            pl.BlockSpec((BN, 9), lambda i: (i, 0)),
            pl.BlockSpec((VOCAB_PAD, H), lambda i: (0, 0)),
            pl.BlockSpec((BN, H), lambda i: (i, 0)),
            pl.BlockSpec((BN, H), lambda i: (i, 0)),
            pl.BlockSpec((H, H), lambda i: (0, 0)),
        ],
        out_specs=[
            pl.BlockSpec((BN, H), lambda i: (i, 0)),
            pl.BlockSpec((BN, 1), lambda i: (i, 0)),
        ],
        out_shape=[
            jax.ShapeDtypeStruct((N, H), jnp.float32),
            jax.ShapeDtypeStruct((N, 1), jnp.float32),
        ],
    )(x, tab, dega, degb, w1)


def _tc_layer_body(acca_ref, accb_ref, g_ref, d_ref, b_ref, w_ref, o_ref):
    d = d_ref[:]
    agg = d * (acca_ref[:] + accb_ref[:] + g_ref[:]) + b_ref[:]
    h = jnp.maximum(agg, 0.0)
    o_ref[:] = d * jnp.dot(h, w_ref[:], preferred_element_type=jnp.float32)


def _tc_layer(acca, accb, g, d, b, w):
    return pl.pallas_call(
        _tc_layer_body,
        grid=(NB,),
        in_specs=[
            pl.BlockSpec((BN, H), lambda i: (i, 0)),
            pl.BlockSpec((BN, H), lambda i: (i, 0)),
            pl.BlockSpec((BN, H), lambda i: (i, 0)),
            pl.BlockSpec((BN, 1), lambda i: (i, 0)),
            pl.BlockSpec((1, H), lambda i: (0, 0)),
            pl.BlockSpec((H, H), lambda i: (0, 0)),
        ],
        out_specs=pl.BlockSpec((BN, H), lambda i: (i, 0)),
        out_shape=jax.ShapeDtypeStruct((N, H), jnp.float32),
    )(acca, accb, g, d, b, w)


def _tc_pool_body(acca_ref, accb_ref, g_ref, d_ref, b_ref, batch_ref,
                  lw_ref, lb_ref, o_ref, sums_ref, cnts_ref):
    i = pl.program_id(0)

    @pl.when(i == 0)
    def _():
        sums_ref[:] = jnp.zeros((G, H), jnp.float32)
        cnts_ref[:] = jnp.zeros((G, 1), jnp.float32)

    d = d_ref[:]
    h4 = d * (acca_ref[:] + accb_ref[:] + g_ref[:]) + b_ref[:]
    iota = lax.broadcasted_iota(jnp.int32, (BN, G), 1)
    p = (iota == batch_ref[:]).astype(jnp.float32)
    sums_ref[:] += lax.dot_general(p, h4, (((0,), (0,)), ((), ())),
                                   preferred_element_type=jnp.float32)
    cnts_ref[:] += lax.dot_general(p, jnp.ones((BN, 1), jnp.float32),
                                   (((0,), (0,)), ((), ())),
                                   preferred_element_type=jnp.float32)

    @pl.when(i == NB - 1)
    def _():
        pooled = sums_ref[:] / jnp.maximum(cnts_ref[:], 1.0)
        o_ref[:] = (jnp.dot(pooled, lw_ref[:], preferred_element_type=jnp.float32)
                    + lb_ref[:])


def _tc_pool(acca, accb, g, d, b, batch, lw, lb):
    return pl.pallas_call(
        _tc_pool_body,
        grid=(NB,),
        in_specs=[
            pl.BlockSpec((BN, H), lambda i: (i, 0)),
            pl.BlockSpec((BN, H), lambda i: (i, 0)),
            pl.BlockSpec((BN, H), lambda i: (i, 0)),
            pl.BlockSpec((BN, 1), lambda i: (i, 0)),
            pl.BlockSpec((1, H), lambda i: (0, 0)),
            pl.BlockSpec((BN, 1), lambda i: (i, 0)),
            pl.BlockSpec((H, TASKS), lambda i: (0, 0)),
            pl.BlockSpec((1, TASKS), lambda i: (0, 0)),
        ],
        out_specs=pl.BlockSpec((G, TASKS), lambda i: (0, 0)),
        out_shape=jax.ShapeDtypeStruct((G, TASKS), jnp.float32),
        scratch_shapes=[
            pltpu.VMEM((G, H), jnp.float32),
            pltpu.VMEM((G, 1), jnp.float32),
        ],
    )(acca, accb, g, d, b, batch, lw, lb)


def kernel(x, edge_index, batch, atom_table, W1, b1, W2, b2, W3, b3,
           lin_W, lin_b):
    x = x.astype(jnp.int32) + jnp.asarray(_OFFSETS, jnp.int32)[None, :]
    row = edge_index[0].astype(jnp.int32)
    col = edge_index[1].astype(jnp.int32)
    pad = EP - E
    row_p = jnp.concatenate([row, jnp.zeros((pad,), jnp.int32)])
    col_p = jnp.concatenate([col, jnp.full((pad,), N, jnp.int32)])
    row_p = row_p.reshape(NW, NCHUNK, CHUNK)
    col_p = col_p.reshape(NW, NCHUNK, CHUNK)

    zeros_h = jnp.zeros((NP, H), jnp.float32)
    ones_h = jnp.ones((CHUNK, H), jnp.float32)
    tab = jnp.pad(atom_table.astype(jnp.float32), ((0, VOCAB_PAD - VOCAB), (0, 0)))

    deg2 = _sc_degree()(col_p, zeros_h, ones_h)
    g1, d = _tc_embed(x, tab, deg2[0], deg2[1], W1)
    scat = _sc_scatter()
    acc1 = scat(g1, row_p, col_p, zeros_h)
    g2 = _tc_layer(acc1[0], acc1[1], g1, d, b1.reshape(1, H), W2)
    acc2 = scat(g2, row_p, col_p, zeros_h)
    g3 = _tc_layer(acc2[0], acc2[1], g2, d, b2.reshape(1, H), W3)
    acc3 = scat(g3, row_p, col_p, zeros_h)
    out = _tc_pool(acc3[0], acc3[1], g3, d, b3.reshape(1, H),
                   batch.reshape(N, 1).astype(jnp.int32),
                   lin_W, lin_b.reshape(1, TASKS))
    return out


# speedup vs baseline: 2.5721x; 2.5721x over previous
import functools

import jax
import jax.numpy as jnp
from jax import lax
from jax.experimental import pallas as pl
from jax.experimental.pallas import tpu as pltpu
from jax.experimental.pallas import tpu_sc as plsc

N = 10000
NP = 10112
H = 128
TASKS = 128
G = 512
E = 320000
CHUNK = 128
NCHUNK = 80
PCH = 40
NW = 32
EP = NW * NCHUNK * CHUNK
VOCAB = 173
VOCAB_PAD = 176
_OFFSETS = (0, 119, 123, 135, 147, 157, 163, 169, 171)
BN = 400
NB = N // BN
RPT = NP // 16


def _mesh():
    return plsc.VectorSubcoreMesh(core_axis_name="c", subcore_axis_name="s",
                                  num_cores=2, num_subcores=16)


def _sc_degree_body(col_hbm, z_hbm, ones_hbm, out_hbm, col_v, ones_v, acc_sh):
    c = lax.axis_index("c")
    s = lax.axis_index("s")
    wid = c * 16 + s
    r0 = s * RPT
    pltpu.sync_copy(z_hbm.at[pl.ds(r0, RPT)], acc_sh.at[pl.ds(r0, RPT)])
    pltpu.sync_copy(col_hbm.at[wid], col_v)
    pltpu.sync_copy(ones_hbm, ones_v)
    plsc.subcore_barrier()

    def body(j, carry):
        pltpu.sync_copy(ones_v, acc_sh.at[col_v.at[j]], add=True)
        return carry

    lax.fori_loop(0, NCHUNK, body, 0)
    plsc.subcore_barrier()
    pltpu.sync_copy(acc_sh.at[pl.ds(r0, RPT)], out_hbm.at[c, pl.ds(r0, RPT)])


@functools.cache
def _sc_degree():
    return pl.kernel(
        _sc_degree_body,
        out_type=jax.ShapeDtypeStruct((2, NP, H), jnp.float32),
        mesh=_mesh(),
        scratch_types=[
            pltpu.VMEM((NCHUNK, CHUNK), jnp.int32),
            pltpu.VMEM((CHUNK, H), jnp.float32),
            pltpu.VMEM_SHARED((NP, H), jnp.float32),
        ],
    )


def _sc_scatter_body(g_hbm, row_hbm, col_hbm, z_hbm, out_hbm,
                     row_v, col_v, msg_a, msg_b, acc_sh, sem_a, sem_b):
    c = lax.axis_index("c")
    s = lax.axis_index("s")
    wid = c * 16 + s
    r0 = s * RPT
    pltpu.sync_copy(z_hbm.at[pl.ds(r0, RPT)], acc_sh.at[pl.ds(r0, RPT)])
    plsc.subcore_barrier()

    def start(j, buf, sem):
        pltpu.async_copy(g_hbm.at[row_v.at[j]], buf, sem)

    def wait(buf, sem):
        pltpu.make_async_copy(g_hbm.at[row_v.at[0]], buf, sem).wait()

    for p in range(NCHUNK // PCH):
        pltpu.sync_copy(row_hbm.at[wid, pl.ds(p * PCH, PCH)], row_v)
        pltpu.sync_copy(col_hbm.at[wid, pl.ds(p * PCH, PCH)], col_v)
        start(0, msg_a, sem_a)

        def body(i, carry):
            wait(msg_a, sem_a)
            start(2 * i + 1, msg_b, sem_b)
            pltpu.sync_copy(msg_a, acc_sh.at[col_v.at[2 * i]], add=True)
            wait(msg_b, sem_b)

            @pl.when(2 * i + 2 < PCH)
            def _():
                start(2 * i + 2, msg_a, sem_a)

            pltpu.sync_copy(msg_b, acc_sh.at[col_v.at[2 * i + 1]], add=True)
            return carry

        lax.fori_loop(0, PCH // 2, body, 0)
    plsc.subcore_barrier()
    pltpu.sync_copy(acc_sh.at[pl.ds(r0, RPT)], out_hbm.at[c, pl.ds(r0, RPT)])


@functools.cache
def _sc_scatter():
    return pl.kernel(
        _sc_scatter_body,
        out_type=jax.ShapeDtypeStruct((2, NP, H), jnp.float32),
        mesh=_mesh(),
        scratch_types=[
            pltpu.VMEM((PCH, CHUNK), jnp.int32),
            pltpu.VMEM((PCH, CHUNK), jnp.int32),
            pltpu.VMEM((CHUNK, H), jnp.float32),
            pltpu.VMEM((CHUNK, H), jnp.float32),
            pltpu.VMEM_SHARED((NP, H), jnp.float32),
            pltpu.SemaphoreType.DMA,
            pltpu.SemaphoreType.DMA,
        ],
    )


def _tc_embed_body(x_ref, tab_ref, dega_ref, degb_ref, w_ref, g_ref, d_ref):
    idx = x_ref[:]
    iota = lax.broadcasted_iota(jnp.int32, (BN, VOCAB_PAD), 1)
    p = jnp.zeros((BN, VOCAB_PAD), jnp.float32)
    for j in range(9):
        p = p + (iota == idx[:, j:j + 1]).astype(jnp.float32)
    h = jnp.dot(p, tab_ref[:], preferred_element_type=jnp.float32)
    deg = dega_ref[:, 0:1] + degb_ref[:, 0:1] + 1.0
    d = lax.rsqrt(deg)
    g_ref[:] = d * jnp.dot(h, w_ref[:], preferred_element_type=jnp.float32)
    d_ref[:] = d


def _tc_embed(x, tab, dega, degb, w1):
    return pl.pallas_call(
        _tc_embed_body,
        grid=(NB,),
        in_specs=[
            pl.BlockSpec((BN, 9), lambda i: (i, 0)),
            pl.BlockSpec((VOCAB_PAD, H), lambda i: (0, 0)),
            pl.BlockSpec((BN, H), lambda i: (i, 0)),
            pl.BlockSpec((BN, H), lambda i: (i, 0)),
            pl.BlockSpec((H, H), lambda i: (0, 0)),
        ],
        out_specs=[
            pl.BlockSpec((BN, H), lambda i: (i, 0)),
            pl.BlockSpec((BN, 1), lambda i: (i, 0)),
        ],
        out_shape=[
            jax.ShapeDtypeStruct((N, H), jnp.float32),
            jax.ShapeDtypeStruct((N, 1), jnp.float32),
        ],
    )(x, tab, dega, degb, w1)


def _tc_layer_body(acca_ref, accb_ref, g_ref, d_ref, b_ref, w_ref, o_ref):
    d = d_ref[:]
    agg = d * (acca_ref[:] + accb_ref[:] + g_ref[:]) + b_ref[:]
    h = jnp.maximum(agg, 0.0)
    o_ref[:] = d * jnp.dot(h, w_ref[:], preferred_element_type=jnp.float32)


def _tc_layer(acca, accb, g, d, b, w):
    return pl.pallas_call(
        _tc_layer_body,
        grid=(NB,),
        in_specs=[
            pl.BlockSpec((BN, H), lambda i: (i, 0)),
            pl.BlockSpec((BN, H), lambda i: (i, 0)),
            pl.BlockSpec((BN, H), lambda i: (i, 0)),
            pl.BlockSpec((BN, 1), lambda i: (i, 0)),
            pl.BlockSpec((1, H), lambda i: (0, 0)),
            pl.BlockSpec((H, H), lambda i: (0, 0)),
        ],
        out_specs=pl.BlockSpec((BN, H), lambda i: (i, 0)),
        out_shape=jax.ShapeDtypeStruct((N, H), jnp.float32),
    )(acca, accb, g, d, b, w)


def _tc_pool_body(acca_ref, accb_ref, g_ref, d_ref, b_ref, batch_ref,
                  lw_ref, lb_ref, o_ref, sums_ref, cnts_ref):
    i = pl.program_id(0)

    @pl.when(i == 0)
    def _():
        sums_ref[:] = jnp.zeros((G, H), jnp.float32)
        cnts_ref[:] = jnp.zeros((G, 1), jnp.float32)

    d = d_ref[:]
    h4 = d * (acca_ref[:] + accb_ref[:] + g_ref[:]) + b_ref[:]
    iota = lax.broadcasted_iota(jnp.int32, (BN, G), 1)
    p = (iota == batch_ref[:]).astype(jnp.float32)
    sums_ref[:] += lax.dot_general(p, h4, (((0,), (0,)), ((), ())),
                                   preferred_element_type=jnp.float32)
    cnts_ref[:] += lax.dot_general(p, jnp.ones((BN, 1), jnp.float32),
                                   (((0,), (0,)), ((), ())),
                                   preferred_element_type=jnp.float32)

    @pl.when(i == NB - 1)
    def _():
        pooled = sums_ref[:] / jnp.maximum(cnts_ref[:], 1.0)
        o_ref[:] = (jnp.dot(pooled, lw_ref[:], preferred_element_type=jnp.float32)
                    + lb_ref[:])


def _tc_pool(acca, accb, g, d, b, batch, lw, lb):
    return pl.pallas_call(
        _tc_pool_body,
        grid=(NB,),
        in_specs=[
            pl.BlockSpec((BN, H), lambda i: (i, 0)),
            pl.BlockSpec((BN, H), lambda i: (i, 0)),
            pl.BlockSpec((BN, H), lambda i: (i, 0)),
            pl.BlockSpec((BN, 1), lambda i: (i, 0)),
            pl.BlockSpec((1, H), lambda i: (0, 0)),
            pl.BlockSpec((BN, 1), lambda i: (i, 0)),
            pl.BlockSpec((H, TASKS), lambda i: (0, 0)),
            pl.BlockSpec((1, TASKS), lambda i: (0, 0)),
        ],
        out_specs=pl.BlockSpec((G, TASKS), lambda i: (0, 0)),
        out_shape=jax.ShapeDtypeStruct((G, TASKS), jnp.float32),
        scratch_shapes=[
            pltpu.VMEM((G, H), jnp.float32),
            pltpu.VMEM((G, 1), jnp.float32),
        ],
    )(acca, accb, g, d, b, batch, lw, lb)


def kernel(x, edge_index, batch, atom_table, W1, b1, W2, b2, W3, b3,
           lin_W, lin_b):
    x = x.astype(jnp.int32) + jnp.asarray(_OFFSETS, jnp.int32)[None, :]
    row = edge_index[0].astype(jnp.int32)
    col = edge_index[1].astype(jnp.int32)
    pad = EP - E
    pad_iota = lax.iota(jnp.int32, pad)
    row_p = jnp.concatenate([row, pad_iota % N])
    col_p = jnp.concatenate([col, N + pad_iota % (NP - N)])
    row_p = row_p.reshape(NW, NCHUNK, CHUNK)
    col_p = col_p.reshape(NW, NCHUNK, CHUNK)

    zeros_h = jnp.zeros((NP, H), jnp.float32)
    ones_h = jnp.ones((CHUNK, H), jnp.float32)
    tab = jnp.pad(atom_table.astype(jnp.float32), ((0, VOCAB_PAD - VOCAB), (0, 0)))

    deg2 = _sc_degree()(col_p, zeros_h, ones_h)
    g1, d = _tc_embed(x, tab, deg2[0], deg2[1], W1)
    scat = _sc_scatter()
    acc1 = scat(g1, row_p, col_p, zeros_h)
    g2 = _tc_layer(acc1[0], acc1[1], g1, d, b1.reshape(1, H), W2)
    acc2 = scat(g2, row_p, col_p, zeros_h)
    g3 = _tc_layer(acc2[0], acc2[1], g2, d, b2.reshape(1, H), W3)
    acc3 = scat(g3, row_p, col_p, zeros_h)
    out = _tc_pool(acc3[0], acc3[1], g3, d, b3.reshape(1, H),
                   batch.reshape(N, 1).astype(jnp.int32),
                   lin_W, lin_b.reshape(1, TASKS))
    return out


# speedup vs baseline: 2.6024x; 1.0118x over previous
import functools

import jax
import jax.numpy as jnp
from jax import lax
from jax.experimental import pallas as pl
from jax.experimental.pallas import tpu as pltpu
from jax.experimental.pallas import tpu_sc as plsc

N = 10000
NP = 10112
H = 128
TASKS = 128
G = 512
E = 320000
CHUNK = 128
NCHUNK = 80
PCH = 40
NW = 32
EP = NW * NCHUNK * CHUNK
VOCAB = 173
VOCAB_PAD = 176
_OFFSETS = (0, 119, 123, 135, 147, 157, 163, 169, 171)
BN = 400
NB = N // BN
RPT = NP // 16


def _mesh():
    return plsc.VectorSubcoreMesh(core_axis_name="c", subcore_axis_name="s",
                                  num_cores=2, num_subcores=16)


def _sc_degree_body(col_hbm, z_hbm, ones_hbm, out_hbm, col_v, ones_v, acc_sh):
    c = lax.axis_index("c")
    s = lax.axis_index("s")
    wid = c * 16 + s
    r0 = s * RPT
    pltpu.sync_copy(z_hbm.at[pl.ds(r0, RPT)], acc_sh.at[pl.ds(r0, RPT)])
    pltpu.sync_copy(col_hbm.at[wid], col_v)
    pltpu.sync_copy(ones_hbm, ones_v)
    plsc.subcore_barrier()

    def body(j, carry):
        pltpu.sync_copy(ones_v, acc_sh.at[col_v.at[j]], add=True)
        return carry

    lax.fori_loop(0, NCHUNK, body, 0)
    plsc.subcore_barrier()
    pltpu.sync_copy(acc_sh.at[pl.ds(r0, RPT)], out_hbm.at[c, pl.ds(r0, RPT)])


@functools.cache
def _sc_degree():
    return pl.kernel(
        _sc_degree_body,
        out_type=jax.ShapeDtypeStruct((2, NP, H), jnp.float32),
        mesh=_mesh(),
        scratch_types=[
            pltpu.VMEM((NCHUNK, CHUNK), jnp.int32),
            pltpu.VMEM((CHUNK, H), jnp.float32),
            pltpu.VMEM_SHARED((NP, H), jnp.float32),
        ],
    )


def _sc_scatter_body(g_hbm, row_hbm, col_hbm, z_hbm, out_hbm,
                     row_v, col_v, msg_a, msg_b, acc_sh, sem_a, sem_b):
    c = lax.axis_index("c")
    s = lax.axis_index("s")
    wid = c * 16 + s
    r0 = s * RPT
    pltpu.sync_copy(z_hbm.at[pl.ds(r0, RPT)], acc_sh.at[pl.ds(r0, RPT)])
    plsc.subcore_barrier()

    def start(j, buf, sem):
        pltpu.async_copy(g_hbm.at[row_v.at[j]], buf, sem)

    def wait(buf, sem):
        pltpu.make_async_copy(g_hbm.at[row_v.at[0]], buf, sem).wait()

    for p in range(NCHUNK // PCH):
        pltpu.sync_copy(row_hbm.at[wid, pl.ds(p * PCH, PCH)], row_v)
        pltpu.sync_copy(col_hbm.at[wid, pl.ds(p * PCH, PCH)], col_v)
        start(0, msg_a, sem_a)

        def body(i, carry):
            wait(msg_a, sem_a)
            start(2 * i + 1, msg_b, sem_b)
            pltpu.sync_copy(msg_a, acc_sh.at[col_v.at[2 * i]], add=True)
            wait(msg_b, sem_b)

            @pl.when(2 * i + 2 < PCH)
            def _():
                start(2 * i + 2, msg_a, sem_a)

            pltpu.sync_copy(msg_b, acc_sh.at[col_v.at[2 * i + 1]], add=True)
            return carry

        lax.fori_loop(0, PCH // 2, body, 0)
    plsc.subcore_barrier()
    pltpu.sync_copy(acc_sh.at[pl.ds(r0, RPT)], out_hbm.at[c, pl.ds(r0, RPT)])


@functools.cache
def _sc_scatter():
    return pl.kernel(
        _sc_scatter_body,
        out_type=jax.ShapeDtypeStruct((2, NP, H), jnp.float32),
        mesh=_mesh(),
        scratch_types=[
            pltpu.VMEM((PCH, CHUNK), jnp.int32),
            pltpu.VMEM((PCH, CHUNK), jnp.int32),
            pltpu.VMEM((CHUNK, H), jnp.float32),
            pltpu.VMEM((CHUNK, H), jnp.float32),
            pltpu.VMEM_SHARED((NP, H), jnp.float32),
            pltpu.SemaphoreType.DMA,
            pltpu.SemaphoreType.DMA,
        ],
    )


def _tc_hembed_body(x_ref, tab_ref, h_ref):
    idx = x_ref[:]
    iota = lax.broadcasted_iota(jnp.int32, (BN, VOCAB_PAD), 1)
    p = jnp.zeros((BN, VOCAB_PAD), jnp.float32)
    for j in range(9):
        p = p + (iota == idx[:, j:j + 1]).astype(jnp.float32)
    h_ref[:] = jnp.dot(p, tab_ref[:], preferred_element_type=jnp.float32)


def _tc_hembed(x, tab):
    return pl.pallas_call(
        _tc_hembed_body,
        grid=(NB,),
        in_specs=[
            pl.BlockSpec((BN, 9), lambda i: (i, 0)),
            pl.BlockSpec((VOCAB_PAD, H), lambda i: (0, 0)),
        ],
        out_specs=pl.BlockSpec((BN, H), lambda i: (i, 0)),
        out_shape=jax.ShapeDtypeStruct((N, H), jnp.float32),
    )(x, tab)


def _tc_g1_body(h_ref, dega_ref, degb_ref, w_ref, g_ref, d_ref):
    deg = dega_ref[:, 0:1] + degb_ref[:, 0:1] + 1.0
    d = lax.rsqrt(deg)
    g_ref[:] = d * jnp.dot(h_ref[:], w_ref[:], preferred_element_type=jnp.float32)
    d_ref[:] = d


def _tc_embed(h, dega, degb, w1):
    return pl.pallas_call(
        _tc_g1_body,
        grid=(NB,),
        in_specs=[
            pl.BlockSpec((BN, H), lambda i: (i, 0)),
            pl.BlockSpec((BN, H), lambda i: (i, 0)),
            pl.BlockSpec((BN, H), lambda i: (i, 0)),
            pl.BlockSpec((H, H), lambda i: (0, 0)),
        ],
        out_specs=[
            pl.BlockSpec((BN, H), lambda i: (i, 0)),
            pl.BlockSpec((BN, 1), lambda i: (i, 0)),
        ],
        out_shape=[
            jax.ShapeDtypeStruct((N, H), jnp.float32),
            jax.ShapeDtypeStruct((N, 1), jnp.float32),
        ],
    )(h, dega, degb, w1)


def _tc_layer_body(acca_ref, accb_ref, g_ref, d_ref, b_ref, w_ref, o_ref):
    d = d_ref[:]
    agg = d * (acca_ref[:] + accb_ref[:] + g_ref[:]) + b_ref[:]
    h = jnp.maximum(agg, 0.0)
    o_ref[:] = d * jnp.dot(h, w_ref[:], preferred_element_type=jnp.float32)


def _tc_layer(acca, accb, g, d, b, w):
    return pl.pallas_call(
        _tc_layer_body,
        grid=(NB,),
        in_specs=[
            pl.BlockSpec((BN, H), lambda i: (i, 0)),
            pl.BlockSpec((BN, H), lambda i: (i, 0)),
            pl.BlockSpec((BN, H), lambda i: (i, 0)),
            pl.BlockSpec((BN, 1), lambda i: (i, 0)),
            pl.BlockSpec((1, H), lambda i: (0, 0)),
            pl.BlockSpec((H, H), lambda i: (0, 0)),
        ],
        out_specs=pl.BlockSpec((BN, H), lambda i: (i, 0)),
        out_shape=jax.ShapeDtypeStruct((N, H), jnp.float32),
    )(acca, accb, g, d, b, w)


def _tc_pool_body(acca_ref, accb_ref, g_ref, d_ref, b_ref, batch_ref,
                  lw_ref, lb_ref, o_ref, sums_ref, cnts_ref):
    i = pl.program_id(0)

    @pl.when(i == 0)
    def _():
        sums_ref[:] = jnp.zeros((G, H), jnp.float32)
        cnts_ref[:] = jnp.zeros((G, 1), jnp.float32)

    d = d_ref[:]
    h4 = d * (acca_ref[:] + accb_ref[:] + g_ref[:]) + b_ref[:]
    iota = lax.broadcasted_iota(jnp.int32, (BN, G), 1)
    p = (iota == batch_ref[:]).astype(jnp.float32)
    sums_ref[:] += lax.dot_general(p, h4, (((0,), (0,)), ((), ())),
                                   preferred_element_type=jnp.float32)
    cnts_ref[:] += lax.dot_general(p, jnp.ones((BN, 1), jnp.float32),
                                   (((0,), (0,)), ((), ())),
                                   preferred_element_type=jnp.float32)

    @pl.when(i == NB - 1)
    def _():
        pooled = sums_ref[:] / jnp.maximum(cnts_ref[:], 1.0)
        o_ref[:] = (jnp.dot(pooled, lw_ref[:], preferred_element_type=jnp.float32)
                    + lb_ref[:])


def _tc_pool(acca, accb, g, d, b, batch, lw, lb):
    return pl.pallas_call(
        _tc_pool_body,
        grid=(NB,),
        in_specs=[
            pl.BlockSpec((BN, H), lambda i: (i, 0)),
            pl.BlockSpec((BN, H), lambda i: (i, 0)),
            pl.BlockSpec((BN, H), lambda i: (i, 0)),
            pl.BlockSpec((BN, 1), lambda i: (i, 0)),
            pl.BlockSpec((1, H), lambda i: (0, 0)),
            pl.BlockSpec((BN, 1), lambda i: (i, 0)),
            pl.BlockSpec((H, TASKS), lambda i: (0, 0)),
            pl.BlockSpec((1, TASKS), lambda i: (0, 0)),
        ],
        out_specs=pl.BlockSpec((G, TASKS), lambda i: (0, 0)),
        out_shape=jax.ShapeDtypeStruct((G, TASKS), jnp.float32),
        scratch_shapes=[
            pltpu.VMEM((G, H), jnp.float32),
            pltpu.VMEM((G, 1), jnp.float32),
        ],
    )(acca, accb, g, d, b, batch, lw, lb)


def kernel(x, edge_index, batch, atom_table, W1, b1, W2, b2, W3, b3,
           lin_W, lin_b):
    x = x.astype(jnp.int32) + jnp.asarray(_OFFSETS, jnp.int32)[None, :]
    row = edge_index[0].astype(jnp.int32)
    col = edge_index[1].astype(jnp.int32)
    pad = EP - E
    pad_iota = lax.iota(jnp.int32, pad)
    row_p = jnp.concatenate([row, pad_iota % N])
    col_p = jnp.concatenate([col, N + pad_iota % (NP - N)])
    row_p = row_p.reshape(NW, NCHUNK, CHUNK)
    col_p = col_p.reshape(NW, NCHUNK, CHUNK)

    zeros_h = jnp.zeros((NP, H), jnp.float32)
    ones_h = jnp.ones((CHUNK, H), jnp.float32)
    tab = jnp.pad(atom_table.astype(jnp.float32), ((0, VOCAB_PAD - VOCAB), (0, 0)))

    h_emb = _tc_hembed(x, tab)
    deg2 = _sc_degree()(col_p, zeros_h, ones_h)
    g1, d = _tc_embed(h_emb, deg2[0], deg2[1], W1)
    scat = _sc_scatter()
    acc1 = scat(g1, row_p, col_p, zeros_h)
    g2 = _tc_layer(acc1[0], acc1[1], g1, d, b1.reshape(1, H), W2)
    acc2 = scat(g2, row_p, col_p, zeros_h)
    g3 = _tc_layer(acc2[0], acc2[1], g2, d, b2.reshape(1, H), W3)
    acc3 = scat(g3, row_p, col_p, zeros_h)
    out = _tc_pool(acc3[0], acc3[1], g3, d, b3.reshape(1, H),
                   batch.reshape(N, 1).astype(jnp.int32),
                   lin_W, lin_b.reshape(1, TASKS))
    return out
